# Initial kernel scaffold; baseline (speedup 1.0000x reference)
import jax
import jax.numpy as jnp
from jax.experimental import pallas as pl


def kernel(node_embeddings, edge_index, edge_weights, W1, b1, g1, be1, W2, b2, g2, be2, hop_coef):
    raise NotImplementedError("write your pallas kernel here")


# speedup vs baseline: 8.1307x; 8.1307x over previous
import functools

import jax
import jax.numpy as jnp
from jax import lax
from jax.experimental import pallas as pl
from jax.experimental.pallas import tpu as pltpu
from jax.experimental.pallas import tpu_sc as plsc

_N, _E, _I, _D = 10000, 320000, 128, 3
_NC, _NS = 2, 16
_NW = _NC * _NS
_EPW = _E // _NW
_C = 80
_NCH = _EPW // _C
_UROWS = 80
_NU = _N // _UROWS


def _table_body(coef_ref, x_ref, out_ref):
    out_ref[...] = x_ref[...] * coef_ref[0]


def _build_table(x, coefs):
    return pl.pallas_call(
        _table_body,
        grid=(_D + 1,),
        in_specs=[
            pl.BlockSpec((1, 1, _I), lambda d: (d, 0, 0)),
            pl.BlockSpec((_N, _I), lambda d: (0, 0)),
        ],
        out_specs=pl.BlockSpec((_N, _I), lambda d: (d, 0)),
        out_shape=jax.ShapeDtypeStruct(((_D + 1) * _N, _I), jnp.float32),
    )(coefs, x)


def _sc_aggregate(table, src, dst, w):
    mesh = plsc.VectorSubcoreMesh(core_axis_name="c", subcore_axis_name="s")

    @functools.partial(
        pl.kernel,
        out_type=jax.ShapeDtypeStruct((_NC * _N, _I), jnp.float32),
        mesh=mesh,
        scratch_types=[
            pltpu.VMEM((_C,), jnp.int32),
            pltpu.VMEM((_C,), jnp.int32),
            pltpu.VMEM((_C,), jnp.int32),
            pltpu.VMEM((_C,), jnp.int32),
            pltpu.VMEM((_C, _I), jnp.float32),
            pltpu.VMEM_SHARED((_N, _I), jnp.float32),
            pltpu.SemaphoreType.DMA,
        ],
    )
    def body(table_hbm, src_hbm, dst_hbm, w_hbm, out_hbm,
             w_v, dst_v, src_v, idx_v, rows_v, accum, sem):
        c = lax.axis_index("c")
        s = lax.axis_index("s")
        wid = s * _NC + c
        n_units = (_NU - 1 - s) // _NS + 1

        def init_unit(k, carry):
            r = pl.multiple_of((s + k * _NS) * _UROWS, 8)
            pltpu.sync_copy(table_hbm.at[pl.ds(r, _UROWS)],
                            accum.at[pl.ds(r, _UROWS)])
            return carry

        lax.fori_loop(0, n_units, init_unit, 0)
        plsc.subcore_barrier()

        base = wid * _EPW

        def chunk(ch, carry):
            off = pl.multiple_of(base + ch * _C, 8)
            pltpu.sync_copy(w_hbm.at[pl.ds(off, _C)], w_v)
            pltpu.sync_copy(dst_hbm.at[pl.ds(off, _C)], dst_v)
            pltpu.sync_copy(src_hbm.at[pl.ds(off, _C)], src_v)
            for j in range(_C // 16):
                sl = pl.ds(j * 16, 16)
                idx_v[sl] = w_v[sl] * _N + dst_v[sl]
            pltpu.async_copy(table_hbm.at[idx_v], rows_v, sem).wait()
            pltpu.sync_copy(rows_v, accum.at[src_v], add=True)
            return carry

        lax.fori_loop(0, _NCH, chunk, 0)
        plsc.subcore_barrier()

        def write_unit(k, carry):
            r = pl.multiple_of((s + k * _NS) * _UROWS, 8)
            pltpu.sync_copy(accum.at[pl.ds(r, _UROWS)],
                            out_hbm.at[pl.ds(c * _N + r, _UROWS)])
            return carry

        lax.fori_loop(0, n_units, write_unit, 0)

    return body(table, src, dst, w)


def _mlp_body(x_ref, parts_ref, w1_ref, b1_ref, g1_ref, be1_ref,
              w2_ref, b2_ref, g2_ref, be2_ref, out_ref):
    combined = x_ref[...] + parts_ref[0] + parts_ref[1]

    def layer(h, w_ref, b_ref, g_ref, be_ref):
        h = lax.dot_general(h, w_ref[...], (((1,), (1,)), ((), ())),
                            preferred_element_type=jnp.float32)
        h = h + b_ref[...]
        mu = jnp.mean(h, axis=0, keepdims=True)
        var = jnp.mean((h - mu) ** 2, axis=0, keepdims=True)
        h = g_ref[...] * (h - mu) / jnp.sqrt(var + 1e-5) + be_ref[...]
        return jnp.maximum(h, 0.0)

    h = layer(combined, w1_ref, b1_ref, g1_ref, be1_ref)
    out_ref[...] = layer(h, w2_ref, b2_ref, g2_ref, be2_ref)


def _mlp(x, parts, W1, b1, g1, be1, W2, b2, g2, be2):
    vecs = [v.reshape(1, _I) for v in (b1, g1, be1, b2, g2, be2)]
    return pl.pallas_call(
        _mlp_body,
        out_shape=jax.ShapeDtypeStruct((_N, _I), jnp.float32),
    )(x, parts, W1, vecs[0], vecs[1], vecs[2], W2, vecs[3], vecs[4], vecs[5])


def kernel(node_embeddings, edge_index, edge_weights,
           W1, b1, g1, be1, W2, b2, g2, be2, hop_coef):
    x = node_embeddings
    coefs = jnp.concatenate([jnp.zeros((1,), jnp.float32), hop_coef])
    table = _build_table(
        x, jnp.broadcast_to(coefs[:, None, None], (_D + 1, 1, _I)))
    parts = _sc_aggregate(table, edge_index[0], edge_index[1], edge_weights)
    return _mlp(x, parts.reshape(_NC, _N, _I),
                W1, b1, g1, be1, W2, b2, g2, be2)


# speedup vs baseline: 19.1624x; 2.3568x over previous
import functools

import jax
import jax.numpy as jnp
from jax import lax
from jax.experimental import pallas as pl
from jax.experimental.pallas import tpu as pltpu
from jax.experimental.pallas import tpu_sc as plsc

_N, _E, _I, _D = 10000, 320000, 128, 3
_NC, _NS = 2, 16
_NW = _NC * _NS
_EPW = _E // _NW
_C = 80
_NCH = _EPW // _C
_UROWS = 80
_NU = _N // _UROWS


def _table_body(coef_ref, x_ref, out_ref):
    out_ref[...] = x_ref[...] * coef_ref[0]


def _build_table(x, coefs):
    return pl.pallas_call(
        _table_body,
        grid=(_D + 1,),
        in_specs=[
            pl.BlockSpec((1, 1, _I), lambda d: (d, 0, 0)),
            pl.BlockSpec((_N, _I), lambda d: (0, 0)),
        ],
        out_specs=pl.BlockSpec((_N, _I), lambda d: (d, 0)),
        out_shape=jax.ShapeDtypeStruct(((_D + 1) * _N, _I), jnp.float32),
    )(coefs, x)


def _idx_body(w_ref, dst_ref, out_ref):
    out_ref[...] = w_ref[...] * _N + dst_ref[...]


def _build_idx(w2d, dst2d):
    return pl.pallas_call(
        _idx_body,
        out_shape=jax.ShapeDtypeStruct(w2d.shape, jnp.int32),
    )(w2d, dst2d)


def _sc_aggregate(table, idx, src):
    mesh = plsc.VectorSubcoreMesh(core_axis_name="c", subcore_axis_name="s")

    @functools.partial(
        pl.kernel,
        out_type=jax.ShapeDtypeStruct((_NC * _N, _I), jnp.float32),
        mesh=mesh,
        scratch_types=[
            pltpu.VMEM((_EPW,), jnp.int32),
            pltpu.VMEM((_C,), jnp.int32),
            pltpu.VMEM((_C,), jnp.int32),
            pltpu.VMEM((_C, _I), jnp.float32),
            pltpu.VMEM((_C, _I), jnp.float32),
            pltpu.VMEM_SHARED((_N, _I), jnp.float32),
            pltpu.SemaphoreType.DMA,
            pltpu.SemaphoreType.DMA,
        ],
    )
    def body(table_hbm, idx_hbm, src_hbm, out_hbm,
             idx_buf, src_a, src_b, bank_a, bank_b, accum, sem_a, sem_b):
        c = lax.axis_index("c")
        s = lax.axis_index("s")
        wid = s * _NC + c
        base = pl.multiple_of(wid * _EPW, 8)
        pltpu.sync_copy(idx_hbm.at[pl.ds(base, _EPW)], idx_buf)

        n_units = (_NU - 1 - s) // _NS + 1

        def init_unit(k, carry):
            r = pl.multiple_of((s + k * _NS) * _UROWS, 8)
            pltpu.sync_copy(table_hbm.at[pl.ds(r, _UROWS)],
                            accum.at[pl.ds(r, _UROWS)])
            return carry

        lax.fori_loop(0, n_units, init_unit, 0)

        def fetch(ch, bank, srcv, sem):
            off = pl.multiple_of(ch * _C, 8)
            rows = pltpu.make_async_copy(
                table_hbm.at[idx_buf.at[pl.ds(off, _C)]], bank, sem)
            srcs = pltpu.make_async_copy(
                src_hbm.at[pl.ds(base + off, _C)], srcv, sem)
            return rows, srcs

        for cp in fetch(0, bank_a, src_a, sem_a):
            cp.start()
        plsc.subcore_barrier()

        def run_chunk(ch, bank_x, src_x, sem_x, bank_y, src_y, sem_y):
            @pl.when(ch < _NCH - 1)
            def _():
                for cp in fetch(ch + 1, bank_y, src_y, sem_y):
                    cp.start()
            for cp in fetch(ch, bank_x, src_x, sem_x):
                cp.wait()
            pltpu.sync_copy(bank_x, accum.at[src_x], add=True)

        def chunk_body(ch, carry):
            is_even = lax.rem(ch, 2) == 0

            @pl.when(is_even)
            def _():
                run_chunk(ch, bank_a, src_a, sem_a, bank_b, src_b, sem_b)

            @pl.when(jnp.logical_not(is_even))
            def _():
                run_chunk(ch, bank_b, src_b, sem_b, bank_a, src_a, sem_a)

            return carry

        lax.fori_loop(0, _NCH, chunk_body, 0)
        plsc.subcore_barrier()

        def write_unit(k, carry):
            r = pl.multiple_of((s + k * _NS) * _UROWS, 8)
            pltpu.sync_copy(accum.at[pl.ds(r, _UROWS)],
                            out_hbm.at[pl.ds(c * _N + r, _UROWS)])
            return carry

        lax.fori_loop(0, n_units, write_unit, 0)

    return body(table, idx, src)


def _mlp_body(x_ref, parts_ref, w1_ref, b1_ref, g1_ref, be1_ref,
              w2_ref, b2_ref, g2_ref, be2_ref, out_ref):
    combined = x_ref[...] + parts_ref[0] + parts_ref[1]

    def layer(h, w_ref, b_ref, g_ref, be_ref):
        h = lax.dot_general(h, w_ref[...], (((1,), (1,)), ((), ())),
                            preferred_element_type=jnp.float32)
        h = h + b_ref[...]
        mu = jnp.mean(h, axis=0, keepdims=True)
        var = jnp.mean((h - mu) ** 2, axis=0, keepdims=True)
        h = g_ref[...] * (h - mu) / jnp.sqrt(var + 1e-5) + be_ref[...]
        return jnp.maximum(h, 0.0)

    h = layer(combined, w1_ref, b1_ref, g1_ref, be1_ref)
    out_ref[...] = layer(h, w2_ref, b2_ref, g2_ref, be2_ref)


def _mlp(x, parts, W1, b1, g1, be1, W2, b2, g2, be2):
    vecs = [v.reshape(1, _I) for v in (b1, g1, be1, b2, g2, be2)]
    return pl.pallas_call(
        _mlp_body,
        out_shape=jax.ShapeDtypeStruct((_N, _I), jnp.float32),
    )(x, parts, W1, vecs[0], vecs[1], vecs[2], W2, vecs[3], vecs[4], vecs[5])


def kernel(node_embeddings, edge_index, edge_weights,
           W1, b1, g1, be1, W2, b2, g2, be2, hop_coef):
    x = node_embeddings
    coefs = jnp.concatenate([jnp.zeros((1,), jnp.float32), hop_coef])
    table = _build_table(
        x, jnp.broadcast_to(coefs[:, None, None], (_D + 1, 1, _I)))
    idx = _build_idx(edge_weights.reshape(_E // _I, _I),
                     edge_index[1].reshape(_E // _I, _I))
    parts = _sc_aggregate(table, idx.reshape(_E), edge_index[0])
    return _mlp(x, parts.reshape(_NC, _N, _I),
                W1, b1, g1, be1, W2, b2, g2, be2)


# speedup vs baseline: 21.8122x; 1.1383x over previous
import functools

import jax
import jax.numpy as jnp
from jax import lax
from jax.experimental import pallas as pl
from jax.experimental.pallas import tpu as pltpu
from jax.experimental.pallas import tpu_sc as plsc

_N, _E, _I, _D = 10000, 320000, 128, 3
_NC, _NS = 2, 16
_NW = _NC * _NS
_EPW = _E // _NW
_C = 80
_NCH = _EPW // _C
_UROWS = 80
_NU = _N // _UROWS


def _table_body(coef_ref, x_ref, out_ref):
    out_ref[...] = x_ref[...] * coef_ref[0]


def _build_table(x, coefs):
    return pl.pallas_call(
        _table_body,
        grid=(_D + 1,),
        in_specs=[
            pl.BlockSpec((1, 1, _I), lambda d: (d, 0, 0)),
            pl.BlockSpec((_N, _I), lambda d: (0, 0)),
        ],
        out_specs=pl.BlockSpec((_N, _I), lambda d: (d, 0)),
        out_shape=jax.ShapeDtypeStruct(((_D + 1) * _N, _I), jnp.float32),
    )(coefs, x)


def _idx_body(w_ref, dst_ref, out_ref):
    out_ref[...] = w_ref[...] * _N + dst_ref[...]


def _build_idx(w2d, dst2d):
    return pl.pallas_call(
        _idx_body,
        out_shape=jax.ShapeDtypeStruct(w2d.shape, jnp.int32),
    )(w2d, dst2d)


def _sc_aggregate(table, idx, src):
    mesh = plsc.VectorSubcoreMesh(core_axis_name="c", subcore_axis_name="s")

    @functools.partial(
        pl.kernel,
        out_type=jax.ShapeDtypeStruct((_NC * _N, _I), jnp.float32),
        mesh=mesh,
        scratch_types=[
            pltpu.VMEM((_EPW,), jnp.int32),
            pltpu.VMEM((_C,), jnp.int32),
            pltpu.VMEM((_C,), jnp.int32),
            pltpu.VMEM((_C,), jnp.int32),
            pltpu.VMEM((_C, _I), jnp.float32),
            pltpu.VMEM((_C, _I), jnp.float32),
            pltpu.VMEM((_C, _I), jnp.float32),
            pltpu.VMEM_SHARED((_N, _I), jnp.float32),
            pltpu.SemaphoreType.DMA,
            pltpu.SemaphoreType.DMA,
            pltpu.SemaphoreType.DMA,
            pltpu.SemaphoreType.DMA,
            pltpu.SemaphoreType.DMA,
            pltpu.SemaphoreType.DMA,
        ],
    )
    def body(table_hbm, idx_hbm, src_hbm, out_hbm,
             idx_buf, src_0, src_1, src_2, bank_0, bank_1, bank_2, accum,
             fsem_0, fsem_1, fsem_2, ssem_0, ssem_1, ssem_2):
        srcs = (src_0, src_1, src_2)
        banks = (bank_0, bank_1, bank_2)
        fsems = (fsem_0, fsem_1, fsem_2)
        ssems = (ssem_0, ssem_1, ssem_2)
        c = lax.axis_index("c")
        s = lax.axis_index("s")
        wid = s * _NC + c
        base = pl.multiple_of(wid * _EPW, 8)
        pltpu.sync_copy(idx_hbm.at[pl.ds(base, _EPW)], idx_buf)

        n_units = (_NU - 1 - s) // _NS + 1

        def init_unit(k, carry):
            r = pl.multiple_of((s + k * _NS) * _UROWS, 8)
            pltpu.sync_copy(table_hbm.at[pl.ds(r, _UROWS)],
                            accum.at[pl.ds(r, _UROWS)])
            return carry

        lax.fori_loop(0, n_units, init_unit, 0)

        def fetch(ch, r):
            off = pl.multiple_of(ch * _C, 8)
            rows = pltpu.make_async_copy(
                table_hbm.at[idx_buf.at[pl.ds(off, _C)]], banks[r], fsems[r])
            sidx = pltpu.make_async_copy(
                src_hbm.at[pl.ds(base + off, _C)], srcs[r], fsems[r])
            return rows, sidx

        def scat(r):
            return pltpu.make_async_copy(banks[r], accum.at[srcs[r]], ssems[r])

        for cp in fetch(0, 0) + fetch(1, 1):
            cp.start()
        plsc.subcore_barrier()

        def run_chunk(ch, r):
            t = (r + 2) % 3

            @pl.when((ch >= 1) & (ch < _NCH - 2))
            def _():
                scat(t).wait()

            @pl.when(ch < _NCH - 2)
            def _():
                for cp in fetch(ch + 2, t):
                    cp.start()

            for cp in fetch(ch, r):
                cp.wait()
            scat(r).start(add=True)

        def chunk_body(ch, carry):
            rr = lax.rem(ch, 3)
            for r in range(3):
                @pl.when(rr == r)
                def _(r=r):
                    run_chunk(ch, r)
            return carry

        lax.fori_loop(0, _NCH, chunk_body, 0)
        for r in ((_NCH - 3) % 3, (_NCH - 2) % 3, (_NCH - 1) % 3):
            scat(r).wait()
        plsc.subcore_barrier()

        def write_unit(k, carry):
            r = pl.multiple_of((s + k * _NS) * _UROWS, 8)
            pltpu.sync_copy(accum.at[pl.ds(r, _UROWS)],
                            out_hbm.at[pl.ds(c * _N + r, _UROWS)])
            return carry

        lax.fori_loop(0, n_units, write_unit, 0)

    return body(table, idx, src)


def _mlp_body(x_ref, parts_ref, w1_ref, b1_ref, g1_ref, be1_ref,
              w2_ref, b2_ref, g2_ref, be2_ref, out_ref):
    combined = x_ref[...] + parts_ref[0] + parts_ref[1]

    def layer(h, w_ref, b_ref, g_ref, be_ref):
        h = lax.dot_general(h, w_ref[...], (((1,), (1,)), ((), ())),
                            preferred_element_type=jnp.float32)
        h = h + b_ref[...]
        mu = jnp.mean(h, axis=0, keepdims=True)
        var = jnp.mean((h - mu) ** 2, axis=0, keepdims=True)
        h = g_ref[...] * (h - mu) / jnp.sqrt(var + 1e-5) + be_ref[...]
        return jnp.maximum(h, 0.0)

    h = layer(combined, w1_ref, b1_ref, g1_ref, be1_ref)
    out_ref[...] = layer(h, w2_ref, b2_ref, g2_ref, be2_ref)


def _mlp(x, parts, W1, b1, g1, be1, W2, b2, g2, be2):
    vecs = [v.reshape(1, _I) for v in (b1, g1, be1, b2, g2, be2)]
    return pl.pallas_call(
        _mlp_body,
        out_shape=jax.ShapeDtypeStruct((_N, _I), jnp.float32),
    )(x, parts, W1, vecs[0], vecs[1], vecs[2], W2, vecs[3], vecs[4], vecs[5])


def kernel(node_embeddings, edge_index, edge_weights,
           W1, b1, g1, be1, W2, b2, g2, be2, hop_coef):
    x = node_embeddings
    coefs = jnp.concatenate([jnp.zeros((1,), jnp.float32), hop_coef])
    table = _build_table(
        x, jnp.broadcast_to(coefs[:, None, None], (_D + 1, 1, _I)))
    idx = _build_idx(edge_weights.reshape(_E // _I, _I),
                     edge_index[1].reshape(_E // _I, _I))
    parts = _sc_aggregate(table, idx.reshape(_E), edge_index[0])
    return _mlp(x, parts.reshape(_NC, _N, _I),
                W1, b1, g1, be1, W2, b2, g2, be2)


# speedup vs baseline: 22.1328x; 1.0147x over previous
import functools

import jax
import jax.numpy as jnp
from jax import lax
from jax.experimental import pallas as pl
from jax.experimental.pallas import tpu as pltpu
from jax.experimental.pallas import tpu_sc as plsc

_N, _E, _I, _D = 10000, 320000, 128, 3
_NC, _NS = 2, 16
_NW = _NC * _NS
_EPW = _E // _NW
_C = 80
_NCH = _EPW // _C
_UROWS = 80
_NU = _N // _UROWS


def _prep_body(coef_ref, x_ref, w_ref, dst_ref, table_ref, idx_ref):
    table_ref[...] = x_ref[...] * coef_ref[0]

    @pl.when(pl.program_id(0) == 0)
    def _():
        idx_ref[...] = w_ref[...] * _N + dst_ref[...]


def _build_table_and_idx(x, coefs, w2d, dst2d):
    return pl.pallas_call(
        _prep_body,
        grid=(_D + 1,),
        in_specs=[
            pl.BlockSpec((1, 1, _I), lambda d: (d, 0, 0)),
            pl.BlockSpec((_N, _I), lambda d: (0, 0)),
            pl.BlockSpec(w2d.shape, lambda d: (0, 0)),
            pl.BlockSpec(w2d.shape, lambda d: (0, 0)),
        ],
        out_specs=[
            pl.BlockSpec((_N, _I), lambda d: (d, 0)),
            pl.BlockSpec(w2d.shape, lambda d: (0, 0)),
        ],
        out_shape=[
            jax.ShapeDtypeStruct(((_D + 1) * _N, _I), jnp.float32),
            jax.ShapeDtypeStruct(w2d.shape, jnp.int32),
        ],
    )(coefs, x, w2d, dst2d)


def _sc_aggregate(table, idx, src):
    mesh = plsc.VectorSubcoreMesh(core_axis_name="c", subcore_axis_name="s")

    @functools.partial(
        pl.kernel,
        out_type=jax.ShapeDtypeStruct((_NC * _N, _I), jnp.float32),
        mesh=mesh,
        scratch_types=[
            pltpu.VMEM((_EPW,), jnp.int32),
            pltpu.VMEM((_C,), jnp.int32),
            pltpu.VMEM((_C,), jnp.int32),
            pltpu.VMEM((_C,), jnp.int32),
            pltpu.VMEM((_C, _I), jnp.float32),
            pltpu.VMEM((_C, _I), jnp.float32),
            pltpu.VMEM((_C, _I), jnp.float32),
            pltpu.VMEM_SHARED((_N, _I), jnp.float32),
            pltpu.SemaphoreType.DMA,
            pltpu.SemaphoreType.DMA,
            pltpu.SemaphoreType.DMA,
            pltpu.SemaphoreType.DMA,
            pltpu.SemaphoreType.DMA,
            pltpu.SemaphoreType.DMA,
        ],
    )
    def body(table_hbm, idx_hbm, src_hbm, out_hbm,
             idx_buf, src_0, src_1, src_2, bank_0, bank_1, bank_2, accum,
             fsem_0, fsem_1, fsem_2, ssem_0, ssem_1, ssem_2):
        srcs = (src_0, src_1, src_2)
        banks = (bank_0, bank_1, bank_2)
        fsems = (fsem_0, fsem_1, fsem_2)
        ssems = (ssem_0, ssem_1, ssem_2)
        c = lax.axis_index("c")
        s = lax.axis_index("s")
        wid = s * _NC + c
        base = pl.multiple_of(wid * _EPW, 8)
        pltpu.sync_copy(idx_hbm.at[pl.ds(base, _EPW)], idx_buf)

        n_units = (_NU - 1 - s) // _NS + 1

        def init_unit(k, carry):
            r = pl.multiple_of((s + k * _NS) * _UROWS, 8)
            pltpu.sync_copy(table_hbm.at[pl.ds(r, _UROWS)],
                            accum.at[pl.ds(r, _UROWS)])
            return carry

        lax.fori_loop(0, n_units, init_unit, 0)

        def fetch(ch, r):
            off = pl.multiple_of(ch * _C, 8)
            rows = pltpu.make_async_copy(
                table_hbm.at[idx_buf.at[pl.ds(off, _C)]], banks[r], fsems[r])
            sidx = pltpu.make_async_copy(
                src_hbm.at[pl.ds(base + off, _C)], srcs[r], fsems[r])
            return rows, sidx

        def scat(r):
            return pltpu.make_async_copy(banks[r], accum.at[srcs[r]], ssems[r])

        for cp in fetch(0, 0) + fetch(1, 1):
            cp.start()
        plsc.subcore_barrier()

        def run_chunk(ch, r):
            t = (r + 2) % 3

            @pl.when((ch >= 1) & (ch < _NCH - 2))
            def _():
                scat(t).wait()

            @pl.when(ch < _NCH - 2)
            def _():
                for cp in fetch(ch + 2, t):
                    cp.start()

            for cp in fetch(ch, r):
                cp.wait()
            scat(r).start(add=True)

        def chunk_body(ch, carry):
            rr = lax.rem(ch, 3)
            for r in range(3):
                @pl.when(rr == r)
                def _(r=r):
                    run_chunk(ch, r)
            return carry

        lax.fori_loop(0, _NCH, chunk_body, 0)
        for r in ((_NCH - 3) % 3, (_NCH - 2) % 3, (_NCH - 1) % 3):
            scat(r).wait()
        plsc.subcore_barrier()

        def write_unit(k, carry):
            r = pl.multiple_of((s + k * _NS) * _UROWS, 8)
            pltpu.sync_copy(accum.at[pl.ds(r, _UROWS)],
                            out_hbm.at[pl.ds(c * _N + r, _UROWS)])
            return carry

        lax.fori_loop(0, n_units, write_unit, 0)

    return body(table, idx, src)


def _mlp_body(x_ref, parts_ref, w1_ref, b1_ref, g1_ref, be1_ref,
              w2_ref, b2_ref, g2_ref, be2_ref, out_ref):
    combined = x_ref[...] + parts_ref[0] + parts_ref[1]

    def layer(h, w_ref, b_ref, g_ref, be_ref):
        h = lax.dot_general(h, w_ref[...], (((1,), (1,)), ((), ())),
                            preferred_element_type=jnp.float32)
        h = h + b_ref[...]
        mu = jnp.mean(h, axis=0, keepdims=True)
        var = jnp.mean((h - mu) ** 2, axis=0, keepdims=True)
        h = g_ref[...] * (h - mu) / jnp.sqrt(var + 1e-5) + be_ref[...]
        return jnp.maximum(h, 0.0)

    h = layer(combined, w1_ref, b1_ref, g1_ref, be1_ref)
    out_ref[...] = layer(h, w2_ref, b2_ref, g2_ref, be2_ref)


def _mlp(x, parts, W1, b1, g1, be1, W2, b2, g2, be2):
    vecs = [v.reshape(1, _I) for v in (b1, g1, be1, b2, g2, be2)]
    return pl.pallas_call(
        _mlp_body,
        out_shape=jax.ShapeDtypeStruct((_N, _I), jnp.float32),
    )(x, parts, W1, vecs[0], vecs[1], vecs[2], W2, vecs[3], vecs[4], vecs[5])


def kernel(node_embeddings, edge_index, edge_weights,
           W1, b1, g1, be1, W2, b2, g2, be2, hop_coef):
    x = node_embeddings
    coefs = jnp.concatenate([jnp.zeros((1,), jnp.float32), hop_coef])
    table, idx = _build_table_and_idx(
        x, jnp.broadcast_to(coefs[:, None, None], (_D + 1, 1, _I)),
        edge_weights.reshape(_E // _I, _I), edge_index[1].reshape(_E // _I, _I))
    parts = _sc_aggregate(table, idx.reshape(_E), edge_index[0])
    return _mlp(x, parts.reshape(_NC, _N, _I),
                W1, b1, g1, be1, W2, b2, g2, be2)


# speedup vs baseline: 22.5541x; 1.0190x over previous
import functools

import jax
import jax.numpy as jnp
from jax import lax
from jax.experimental import pallas as pl
from jax.experimental.pallas import tpu as pltpu
from jax.experimental.pallas import tpu_sc as plsc

_N, _E, _I, _D = 10000, 320000, 128, 3
_NC, _NS = 2, 16
_NW = _NC * _NS
_EPW = _E // _NW
_C = 80
_NCH = _EPW // _C
_UROWS = 80
_NU = _N // _UROWS


def _prep_body(coef_ref, x_ref, w_ref, dst_ref, table_ref, idx_ref):
    table_ref[...] = x_ref[...] * coef_ref[0]

    @pl.when(pl.program_id(0) == 0)
    def _():
        idx_ref[...] = w_ref[...] * _N + dst_ref[...]


def _build_table_and_idx(x, coefs, w2d, dst2d):
    return pl.pallas_call(
        _prep_body,
        grid=(_D + 1,),
        in_specs=[
            pl.BlockSpec((1, 1, _I), lambda d: (d, 0, 0)),
            pl.BlockSpec((_N, _I), lambda d: (0, 0)),
            pl.BlockSpec(w2d.shape, lambda d: (0, 0)),
            pl.BlockSpec(w2d.shape, lambda d: (0, 0)),
        ],
        out_specs=[
            pl.BlockSpec((_N, _I), lambda d: (d, 0)),
            pl.BlockSpec(w2d.shape, lambda d: (0, 0)),
        ],
        out_shape=[
            jax.ShapeDtypeStruct(((_D + 1) * _N, _I), jnp.float32),
            jax.ShapeDtypeStruct(w2d.shape, jnp.int32),
        ],
    )(coefs, x, w2d, dst2d)


def _sc_aggregate(table, idx, src):
    mesh = plsc.VectorSubcoreMesh(core_axis_name="c", subcore_axis_name="s")

    @functools.partial(
        pl.kernel,
        out_type=jax.ShapeDtypeStruct((_NC * _N, _I), jnp.float32),
        mesh=mesh,
        scratch_types=[
            pltpu.VMEM((_EPW,), jnp.int32),
            pltpu.VMEM((_C,), jnp.int32),
            pltpu.VMEM((_C,), jnp.int32),
            pltpu.VMEM((_C,), jnp.int32),
            pltpu.VMEM((_C, _I), jnp.float32),
            pltpu.VMEM((_C, _I), jnp.float32),
            pltpu.VMEM((_C, _I), jnp.float32),
            pltpu.VMEM_SHARED((_N, _I), jnp.float32),
            pltpu.SemaphoreType.DMA,
            pltpu.SemaphoreType.DMA,
            pltpu.SemaphoreType.DMA,
            pltpu.SemaphoreType.DMA,
            pltpu.SemaphoreType.DMA,
            pltpu.SemaphoreType.DMA,
            pltpu.SemaphoreType.DMA,
        ],
    )
    def body(table_hbm, idx_hbm, src_hbm, out_hbm,
             idx_buf, src_0, src_1, src_2, bank_0, bank_1, bank_2, accum,
             fsem_0, fsem_1, fsem_2, ssem_0, ssem_1, ssem_2, isem):
        srcs = (src_0, src_1, src_2)
        banks = (bank_0, bank_1, bank_2)
        fsems = (fsem_0, fsem_1, fsem_2)
        ssems = (ssem_0, ssem_1, ssem_2)
        c = lax.axis_index("c")
        s = lax.axis_index("s")
        wid = s * _NC + c
        base = pl.multiple_of(wid * _EPW, 8)
        pltpu.sync_copy(idx_hbm.at[pl.ds(base, _EPW)], idx_buf)

        n_units = (_NU - 1 - s) // _NS + 1

        def init_unit(k, carry):
            r = pl.multiple_of((s + k * _NS) * _UROWS, 8)
            return pltpu.make_async_copy(table_hbm.at[pl.ds(r, _UROWS)],
                                         accum.at[pl.ds(r, _UROWS)], isem)

        def fetch(ch, r):
            off = pl.multiple_of(ch * _C, 8)
            rows = pltpu.make_async_copy(
                table_hbm.at[idx_buf.at[pl.ds(off, _C)]], banks[r], fsems[r])
            sidx = pltpu.make_async_copy(
                src_hbm.at[pl.ds(base + off, _C)], srcs[r], fsems[r])
            return rows, sidx

        def scat(r):
            return pltpu.make_async_copy(banks[r], accum.at[srcs[r]], ssems[r])

        for cp in fetch(0, 0) + fetch(1, 1):
            cp.start()
        lax.fori_loop(0, n_units,
                      lambda k, c: (init_unit(k, c).start(), c)[1], 0)
        lax.fori_loop(0, n_units,
                      lambda k, c: (init_unit(k, c).wait(), c)[1], 0)
        plsc.subcore_barrier()

        def run_chunk(ch, r):
            t = (r + 2) % 3

            @pl.when((ch >= 1) & (ch < _NCH - 2))
            def _():
                scat(t).wait()

            @pl.when(ch < _NCH - 2)
            def _():
                for cp in fetch(ch + 2, t):
                    cp.start()

            for cp in fetch(ch, r):
                cp.wait()
            scat(r).start(add=True)

        def chunk_body(ch, carry):
            rr = lax.rem(ch, 3)
            for r in range(3):
                @pl.when(rr == r)
                def _(r=r):
                    run_chunk(ch, r)
            return carry

        lax.fori_loop(0, _NCH, chunk_body, 0)
        for r in ((_NCH - 3) % 3, (_NCH - 2) % 3, (_NCH - 1) % 3):
            scat(r).wait()
        plsc.subcore_barrier()

        def write_unit(k, carry):
            r = pl.multiple_of((s + k * _NS) * _UROWS, 8)
            return pltpu.make_async_copy(accum.at[pl.ds(r, _UROWS)],
                                         out_hbm.at[pl.ds(c * _N + r, _UROWS)],
                                         isem)

        lax.fori_loop(0, n_units,
                      lambda k, c: (write_unit(k, c).start(), c)[1], 0)
        lax.fori_loop(0, n_units,
                      lambda k, c: (write_unit(k, c).wait(), c)[1], 0)

    return body(table, idx, src)


def _mlp_body(x_ref, parts_ref, w1_ref, b1_ref, g1_ref, be1_ref,
              w2_ref, b2_ref, g2_ref, be2_ref, out_ref):
    combined = x_ref[...] + parts_ref[0] + parts_ref[1]

    def layer(h, w_ref, b_ref, g_ref, be_ref):
        h = lax.dot_general(h, w_ref[...], (((1,), (1,)), ((), ())),
                            preferred_element_type=jnp.float32)
        h = h + b_ref[...]
        mu = jnp.mean(h, axis=0, keepdims=True)
        var = jnp.mean((h - mu) ** 2, axis=0, keepdims=True)
        h = g_ref[...] * (h - mu) / jnp.sqrt(var + 1e-5) + be_ref[...]
        return jnp.maximum(h, 0.0)

    h = layer(combined, w1_ref, b1_ref, g1_ref, be1_ref)
    out_ref[...] = layer(h, w2_ref, b2_ref, g2_ref, be2_ref)


def _mlp(x, parts, W1, b1, g1, be1, W2, b2, g2, be2):
    vecs = [v.reshape(1, _I) for v in (b1, g1, be1, b2, g2, be2)]
    return pl.pallas_call(
        _mlp_body,
        out_shape=jax.ShapeDtypeStruct((_N, _I), jnp.float32),
    )(x, parts, W1, vecs[0], vecs[1], vecs[2], W2, vecs[3], vecs[4], vecs[5])


def kernel(node_embeddings, edge_index, edge_weights,
           W1, b1, g1, be1, W2, b2, g2, be2, hop_coef):
    x = node_embeddings
    coefs = jnp.concatenate([jnp.zeros((1,), jnp.float32), hop_coef])
    table, idx = _build_table_and_idx(
        x, jnp.broadcast_to(coefs[:, None, None], (_D + 1, 1, _I)),
        edge_weights.reshape(_E // _I, _I), edge_index[1].reshape(_E // _I, _I))
    parts = _sc_aggregate(table, idx.reshape(_E), edge_index[0])
    return _mlp(x, parts.reshape(_NC, _N, _I),
                W1, b1, g1, be1, W2, b2, g2, be2)


# speedup vs baseline: 22.5978x; 1.0019x over previous
import functools

import jax
import jax.numpy as jnp
from jax import lax
from jax.experimental import pallas as pl
from jax.experimental.pallas import tpu as pltpu
from jax.experimental.pallas import tpu_sc as plsc

_N, _E, _I, _D = 10000, 320000, 128, 3
_NC, _NS = 2, 16
_NW = _NC * _NS
_EPW = _E // _NW
_C = 80
_NCH = _EPW // _C
_UROWS = 80
_NU = _N // _UROWS


def _prep_body(coef_ref, x_ref, w_ref, dst_ref, table_ref, idx_ref):
    table_ref[...] = x_ref[...] * coef_ref[0]

    @pl.when(pl.program_id(0) == 0)
    def _():
        idx_ref[...] = w_ref[...] * _N + dst_ref[...]


def _build_table_and_idx(x, coefs, w2d, dst2d):
    return pl.pallas_call(
        _prep_body,
        grid=(_D + 1,),
        in_specs=[
            pl.BlockSpec((1, 1, _I), lambda d: (d, 0, 0)),
            pl.BlockSpec((_N, _I), lambda d: (0, 0)),
            pl.BlockSpec(w2d.shape, lambda d: (0, 0)),
            pl.BlockSpec(w2d.shape, lambda d: (0, 0)),
        ],
        out_specs=[
            pl.BlockSpec((_N, _I), lambda d: (d, 0)),
            pl.BlockSpec(w2d.shape, lambda d: (0, 0)),
        ],
        out_shape=[
            jax.ShapeDtypeStruct(((_D + 1) * _N, _I), jnp.float32),
            jax.ShapeDtypeStruct(w2d.shape, jnp.int32),
        ],
    )(coefs, x, w2d, dst2d)


def _sc_aggregate(table, idx, src):
    mesh = plsc.VectorSubcoreMesh(core_axis_name="c", subcore_axis_name="s")

    @functools.partial(
        pl.kernel,
        out_type=jax.ShapeDtypeStruct((_NC * _N, _I), jnp.float32),
        mesh=mesh,
        scratch_types=[
            pltpu.VMEM((_EPW,), jnp.int32),
            pltpu.VMEM((_C,), jnp.int32),
            pltpu.VMEM((_C,), jnp.int32),
            pltpu.VMEM((_C,), jnp.int32),
            pltpu.VMEM((_C, _I), jnp.float32),
            pltpu.VMEM((_C, _I), jnp.float32),
            pltpu.VMEM((_C, _I), jnp.float32),
            pltpu.VMEM_SHARED((_N, _I), jnp.float32),
            pltpu.SemaphoreType.DMA,
            pltpu.SemaphoreType.DMA,
            pltpu.SemaphoreType.DMA,
            pltpu.SemaphoreType.DMA,
            pltpu.SemaphoreType.DMA,
            pltpu.SemaphoreType.DMA,
            pltpu.SemaphoreType.DMA,
        ],
    )
    def body(table_hbm, idx_hbm, src_hbm, out_hbm,
             idx_buf, src_0, src_1, src_2, bank_0, bank_1, bank_2, accum,
             fsem_0, fsem_1, fsem_2, ssem_0, ssem_1, ssem_2, isem):
        srcs = (src_0, src_1, src_2)
        banks = (bank_0, bank_1, bank_2)
        fsems = (fsem_0, fsem_1, fsem_2)
        ssems = (ssem_0, ssem_1, ssem_2)
        c = lax.axis_index("c")
        s = lax.axis_index("s")
        wid = s * _NC + c
        base = pl.multiple_of(wid * _EPW, 8)
        pltpu.sync_copy(idx_hbm.at[pl.ds(base, _EPW)], idx_buf)

        n_units = (_NU - 1 - s) // _NS + 1

        def init_unit(k, carry):
            r = pl.multiple_of((s + k * _NS) * _UROWS, 8)
            return pltpu.make_async_copy(table_hbm.at[pl.ds(r, _UROWS)],
                                         accum.at[pl.ds(r, _UROWS)], isem)

        def fetch(ch, r):
            off = pl.multiple_of(ch * _C, 8)
            rows = pltpu.make_async_copy(
                table_hbm.at[idx_buf.at[pl.ds(off, _C)]], banks[r], fsems[r])
            sidx = pltpu.make_async_copy(
                src_hbm.at[pl.ds(base + off, _C)], srcs[r], fsems[r])
            return rows, sidx

        def scat(r):
            return pltpu.make_async_copy(banks[r], accum.at[srcs[r]], ssems[r])

        for cp in fetch(0, 0) + fetch(1, 1):
            cp.start()
        lax.fori_loop(0, n_units,
                      lambda k, c: (init_unit(k, c).start(), c)[1], 0)
        lax.fori_loop(0, n_units,
                      lambda k, c: (init_unit(k, c).wait(), c)[1], 0)
        plsc.subcore_barrier()

        def run_chunk(ch, r):
            t = (r + 2) % 3

            @pl.when((ch >= 1) & (ch < _NCH - 2))
            def _():
                scat(t).wait()

            @pl.when(ch < _NCH - 2)
            def _():
                for cp in fetch(ch + 2, t):
                    cp.start()

            for cp in fetch(ch, r):
                cp.wait()
            scat(r).start(add=True)

        def triple_body(g, carry):
            for r in range(3):
                run_chunk(g * 3 + r, r)
            return carry

        lax.fori_loop(0, _NCH // 3, triple_body, 0)
        for ch in range(_NCH - _NCH % 3, _NCH):
            for cp in fetch(ch, ch % 3):
                cp.wait()
            scat(ch % 3).start(add=True)
        for r in ((_NCH - 3) % 3, (_NCH - 2) % 3, (_NCH - 1) % 3):
            scat(r).wait()
        plsc.subcore_barrier()

        def write_unit(k, carry):
            r = pl.multiple_of((s + k * _NS) * _UROWS, 8)
            return pltpu.make_async_copy(accum.at[pl.ds(r, _UROWS)],
                                         out_hbm.at[pl.ds(c * _N + r, _UROWS)],
                                         isem)

        lax.fori_loop(0, n_units,
                      lambda k, c: (write_unit(k, c).start(), c)[1], 0)
        lax.fori_loop(0, n_units,
                      lambda k, c: (write_unit(k, c).wait(), c)[1], 0)

    return body(table, idx, src)


def _mlp_body(x_ref, parts_ref, w1_ref, b1_ref, g1_ref, be1_ref,
              w2_ref, b2_ref, g2_ref, be2_ref, out_ref):
    combined = x_ref[...] + parts_ref[0] + parts_ref[1]

    def layer(h, w_ref, b_ref, g_ref, be_ref):
        h = lax.dot_general(h, w_ref[...], (((1,), (1,)), ((), ())),
                            preferred_element_type=jnp.float32)
        h = h + b_ref[...]
        mu = jnp.mean(h, axis=0, keepdims=True)
        var = jnp.mean((h - mu) ** 2, axis=0, keepdims=True)
        h = g_ref[...] * (h - mu) / jnp.sqrt(var + 1e-5) + be_ref[...]
        return jnp.maximum(h, 0.0)

    h = layer(combined, w1_ref, b1_ref, g1_ref, be1_ref)
    out_ref[...] = layer(h, w2_ref, b2_ref, g2_ref, be2_ref)


def _mlp(x, parts, W1, b1, g1, be1, W2, b2, g2, be2):
    vecs = [v.reshape(1, _I) for v in (b1, g1, be1, b2, g2, be2)]
    return pl.pallas_call(
        _mlp_body,
        out_shape=jax.ShapeDtypeStruct((_N, _I), jnp.float32),
    )(x, parts, W1, vecs[0], vecs[1], vecs[2], W2, vecs[3], vecs[4], vecs[5])


def kernel(node_embeddings, edge_index, edge_weights,
           W1, b1, g1, be1, W2, b2, g2, be2, hop_coef):
    x = node_embeddings
    coefs = jnp.concatenate([jnp.zeros((1,), jnp.float32), hop_coef])
    table, idx = _build_table_and_idx(
        x, jnp.broadcast_to(coefs[:, None, None], (_D + 1, 1, _I)),
        edge_weights.reshape(_E // _I, _I), edge_index[1].reshape(_E // _I, _I))
    parts = _sc_aggregate(table, idx.reshape(_E), edge_index[0])
    return _mlp(x, parts.reshape(_NC, _N, _I),
                W1, b1, g1, be1, W2, b2, g2, be2)
